# baseline (speedup 1.0000x reference)
import functools

import numpy as np
import jax
import jax.numpy as jnp
from jax import lax
from jax.experimental import pallas as pl
from jax.experimental.pallas import tpu as pltpu
from jax.experimental.pallas import tpu_sc as plsc

N = 10000
E = 320000
D = 128
DE = 16

NC = 2
NS = 16
NW = NC * NS
EW = E // NW
CH = 80
NCH = EW // CH
NSTG = 5
CPS = NCH // NSTG
NP_ = 10240
NPS = NP_ // NS

_IDX_A = np.concatenate([np.arange(32 * g, 32 * g + 16) for g in range(4)])
_IDX_B = _IDX_A + 16

_sc_mesh = plsc.VectorSubcoreMesh(core_axis_name="c", subcore_axis_name="s")


def _msgpass_body(x_hbm, ep_hbm, src_hbm, dst_hbm, zeros_hbm, out_hbm,
                  src_v, dst_v, ep0, ep1, rows0, rows1, agg_sh,
                  sem_e0, sem_e1, sem_g0, sem_g1, sem_s0, sem_s1):
    c = lax.axis_index("c")
    s = lax.axis_index("s")
    w = c * NS + s
    pltpu.sync_copy(zeros_hbm, agg_sh.at[pl.ds(s * NPS, NPS)])
    plsc.subcore_barrier()

    def compute(ep_v, rows_v):
        shift16 = jnp.full((16,), 16, jnp.int32)
        mask_hi = jnp.full((16,), -65536, jnp.int32)

        def row(r4, carry2):
            for u in range(4):
                r = r4 * 4 + u
                for g in range(4):
                    epw = ep_v[r, pl.ds(16 * g, 16)]
                    lo = lax.bitcast_convert_type(
                        lax.shift_left(epw, shift16), jnp.float32)
                    hi = lax.bitcast_convert_type(
                        lax.bitwise_and(epw, mask_hi), jnp.float32)
                    sl_lo = pl.ds(32 * g, 16)
                    sl_hi = pl.ds(32 * g + 16, 16)
                    rows_v[r, sl_lo] = jnp.maximum(rows_v[r, sl_lo] + lo, 0.0)
                    rows_v[r, sl_hi] = jnp.maximum(rows_v[r, sl_hi] + hi, 0.0)
            return carry2

        lax.fori_loop(0, CH // 4, row, 0)

    def load(jg, jl, ep_v, rows_v, sem_e, sem_g):
        pltpu.async_copy(ep_hbm.at[w, jg], ep_v, sem_e)
        pltpu.async_copy(x_hbm.at[src_v.at[jl]], rows_v, sem_g)

    def wait_load(jg, jl, ep_v, rows_v, sem_e, sem_g):
        pltpu.make_async_copy(ep_hbm.at[w, jg], ep_v, sem_e).wait()
        pltpu.make_async_copy(x_hbm.at[src_v.at[jl]], rows_v, sem_g).wait()

    def scatter(jl, msg_v, sem_s):
        pltpu.async_copy(msg_v, agg_sh.at[dst_v.at[jl]], sem_s, add=True)

    def wait_scatter(jl, msg_v, sem_s):
        pltpu.make_async_copy(msg_v, agg_sh.at[dst_v.at[jl]], sem_s).wait()

    def stage(g, carry0):
        pltpu.sync_copy(src_hbm.at[w, g], src_v)
        pltpu.sync_copy(dst_hbm.at[w, g], dst_v)
        base = g * CPS
        load(base, 0, ep0, rows0, sem_e0, sem_g0)

        def pair(i, carry):
            j0 = 2 * i
            j1 = j0 + 1

            @pl.when(i > 0)
            def _():
                wait_scatter(j0 - 1, rows1, sem_s1)

            load(base + j1, j1, ep1, rows1, sem_e1, sem_g1)
            wait_load(base + j0, j0, ep0, rows0, sem_e0, sem_g0)
            compute(ep0, rows0)
            scatter(j0, rows0, sem_s0)

            @pl.when(i < CPS // 2)
            def _():
                wait_scatter(j0, rows0, sem_s0)
                load(base + j0 + 2, j0 + 2, ep0, rows0, sem_e0, sem_g0)

            wait_load(base + j1, j1, ep1, rows1, sem_e1, sem_g1)
            compute(ep1, rows1)
            scatter(j1, rows1, sem_s1)
            return carry

        lax.fori_loop(0, CPS // 2, pair, 0)
        jt = CPS - 1
        wait_load(base + jt, jt, ep0, rows0, sem_e0, sem_g0)
        wait_scatter(jt - 1, rows1, sem_s1)
        compute(ep0, rows0)
        scatter(jt, rows0, sem_s0)
        wait_scatter(jt, rows0, sem_s0)
        return carry0

    lax.fori_loop(0, NSTG, stage, 0)
    plsc.subcore_barrier()
    pltpu.sync_copy(agg_sh.at[pl.ds(s * NPS, NPS)],
                    out_hbm.at[c, pl.ds(s * NPS, NPS)])


_msgpass = pl.kernel(
    _msgpass_body,
    out_type=jax.ShapeDtypeStruct((NC, NP_, D), jnp.float32),
    mesh=_sc_mesh,
    scratch_types=[
        pltpu.VMEM((CPS, CH), jnp.int32),
        pltpu.VMEM((CPS, CH), jnp.int32),
        pltpu.VMEM((CH, D // 2), jnp.int32),
        pltpu.VMEM((CH, D // 2), jnp.int32),
        pltpu.VMEM((CH, D), jnp.float32),
        pltpu.VMEM((CH, D), jnp.float32),
        pltpu.VMEM_SHARED((NP_, D), jnp.float32),
        pltpu.SemaphoreType.DMA,
        pltpu.SemaphoreType.DMA,
        pltpu.SemaphoreType.DMA,
        pltpu.SemaphoreType.DMA,
        pltpu.SemaphoreType.DMA,
        pltpu.SemaphoreType.DMA,
    ],
)


EB = 6400
_DN_T = (((0,), (0,)), ((), ()))


def _pack_words(a, b):
    half = jnp.uint32(0x8000)
    au = lax.bitcast_convert_type(a, jnp.uint32) + half
    bu = lax.bitcast_convert_type(b, jnp.uint32) + half
    w = (au >> 16) | (bu & jnp.uint32(0xFFFF0000))
    return lax.bitcast_convert_type(w, jnp.int32)


def _eproj0_body(eat_ref, Wear, bear, Webr, bebr, Weu0r, beu0r,
                 p0_ref, ea1_ref):
    f32 = jnp.float32
    eat = eat_ref[...]
    a = lax.dot_general(eat, Wear[...], _DN_T,
                        preferred_element_type=f32) + bear[...]
    b = lax.dot_general(eat, Webr[...], _DN_T,
                        preferred_element_type=f32) + bebr[...]
    p0_ref[...] = _pack_words(a, b)
    ea1_ref[...] = jnp.maximum(
        lax.dot_general(eat, Weu0r[...], _DN_T, preferred_element_type=f32)
        + beu0r[...], 0.0)


def _eprojN_body(ea_ref, Wear, bear, Webr, bebr, Weur, beur, p_ref, ean_ref):
    f32 = jnp.float32
    ea = ea_ref[...]
    a = jnp.dot(ea, Wear[...], preferred_element_type=f32) + bear[...]
    b = jnp.dot(ea, Webr[...], preferred_element_type=f32) + bebr[...]
    p_ref[...] = _pack_words(a, b)
    ean_ref[...] = jnp.maximum(
        jnp.dot(ea, Weur[...], preferred_element_type=f32) + beur[...], 0.0)


def _eproj_last_body(ea_ref, Wear, bear, Webr, bebr, p_ref):
    f32 = jnp.float32
    ea = ea_ref[...]
    a = jnp.dot(ea, Wear[...], preferred_element_type=f32) + bear[...]
    b = jnp.dot(ea, Webr[...], preferred_element_type=f32) + bebr[...]
    p_ref[...] = _pack_words(a, b)


def _wspec(shape):
    return pl.BlockSpec(shape, lambda i: (0, 0))


def _eproj_call(ea, We0, be0, Weu0, beu0, We1, be1, Weu1, beu1, We2, be2):
    grid = (E // EB,)
    DH = D // 2
    wsplit = lambda We, be: (We[:, _IDX_A], be[_IDX_A].reshape(1, DH),
                             We[:, _IDX_B], be[_IDX_B].reshape(1, DH))
    p0, ea1 = pl.pallas_call(
        _eproj0_body,
        grid=grid,
        in_specs=[
            pl.BlockSpec((DE, EB), lambda i: (0, i)),
            _wspec((DE, DH)), _wspec((1, DH)),
            _wspec((DE, DH)), _wspec((1, DH)),
            _wspec((DE, DE)), _wspec((1, DE)),
        ],
        out_specs=[pl.BlockSpec((EB, DH), lambda i: (i, 0)),
                   pl.BlockSpec((EB, DE), lambda i: (i, 0))],
        out_shape=[jax.ShapeDtypeStruct((E, DH), jnp.int32),
                   jax.ShapeDtypeStruct((E, DE), jnp.float32)],
    )(ea.T, *wsplit(We0, be0), Weu0, beu0.reshape(1, DE))
    p1, ea2 = pl.pallas_call(
        _eprojN_body,
        grid=grid,
        in_specs=[
            pl.BlockSpec((EB, DE), lambda i: (i, 0)),
            _wspec((DE, DH)), _wspec((1, DH)),
            _wspec((DE, DH)), _wspec((1, DH)),
            _wspec((DE, DE)), _wspec((1, DE)),
        ],
        out_specs=[pl.BlockSpec((EB, DH), lambda i: (i, 0)),
                   pl.BlockSpec((EB, DE), lambda i: (i, 0))],
        out_shape=[jax.ShapeDtypeStruct((E, DH), jnp.int32),
                   jax.ShapeDtypeStruct((E, DE), jnp.float32)],
    )(ea1, *wsplit(We1, be1), Weu1, beu1.reshape(1, DE))
    p2 = pl.pallas_call(
        _eproj_last_body,
        grid=grid,
        in_specs=[
            pl.BlockSpec((EB, DE), lambda i: (i, 0)),
            _wspec((DE, DH)), _wspec((1, DH)),
            _wspec((DE, DH)), _wspec((1, DH)),
        ],
        out_specs=pl.BlockSpec((EB, DH), lambda i: (i, 0)),
        out_shape=jax.ShapeDtypeStruct((E, DH), jnp.int32),
    )(ea2, *wsplit(We2, be2))
    return p0, p1, p2


NB = 2000


def _mlp_body(x_ref, pa_ref, pb_ref, W1r, b1r, W2r, b2r, xo_ref, xg_ref):
    f32 = jnp.float32
    i = pl.program_id(0)
    h = x_ref[...] + pa_ref[0] + pb_ref[0]
    t = jnp.maximum(jnp.dot(h, W1r[...], preferred_element_type=f32) + b1r[...],
                    0.0)
    xo = jnp.dot(t, W2r[...], preferred_element_type=f32) + b2r[...]
    xo_ref[...] = xo
    col = jnp.sum(xo, axis=0, keepdims=True)

    @pl.when(i == 0)
    def _():
        xg_ref[...] = col

    @pl.when(i > 0)
    def _():
        xg_ref[...] = xg_ref[...] + col


def _mlp_call(x, parts, W1, b1, W2, b2):
    wspec = lambda shape: pl.BlockSpec(shape, lambda i: (0, 0))
    return pl.pallas_call(
        _mlp_body,
        grid=(N // NB,),
        in_specs=[
            pl.BlockSpec((NB, D), lambda i: (i, 0)),
            pl.BlockSpec((1, NB, D), lambda i: (0, i, 0)),
            pl.BlockSpec((1, NB, D), lambda i: (1, i, 0)),
            wspec((D, D)), wspec((1, D)),
            wspec((D, D)), wspec((1, D)),
        ],
        out_specs=[
            pl.BlockSpec((NB, D), lambda i: (i, 0)),
            pl.BlockSpec((1, D), lambda i: (0, 0)),
        ],
        out_shape=[
            jax.ShapeDtypeStruct((N, D), jnp.float32),
            jax.ShapeDtypeStruct((1, D), jnp.float32),
        ],
    )(x, parts, parts, W1, b1.reshape(1, D), W2, b2.reshape(1, D))


def _final_body(g0, g1, g2, wl, bl, out_ref):
    acc = (jnp.sum(g0[...] * wl[0:1, :]) + jnp.sum(g1[...] * wl[1:2, :])
           + jnp.sum(g2[...] * wl[2:3, :]) + bl[0, 0])
    out_ref[...] = jnp.reshape(acc, (1, 1))


_final = pl.pallas_call(
    _final_body,
    out_shape=jax.ShapeDtypeStruct((1, 1), jnp.float32),
)


def kernel(x, edge_index, edge_attr,
           We0, be0, W1_0, b1_0, W2_0, b2_0, Weu0, beu0,
           We1, be1, W1_1, b1_1, W2_1, b2_1, Weu1, beu1,
           We2, be2, W1_2, b1_2, W2_2, b2_2, Weu2, beu2,
           Wlin, blin):
    src3 = edge_index[0].reshape(NW, NSTG, CPS, CH)
    dst3 = edge_index[1].reshape(NW, NSTG, CPS, CH)
    zeros = jnp.zeros((NPS, D), jnp.float32)

    p0, p1, p2 = _eproj_call(edge_attr, We0, be0, Weu0, beu0,
                             We1, be1, Weu1, beu1, We2, be2)
    projs = [p.reshape(NW, NCH, CH, D // 2) for p in (p0, p1, p2)]
    mlps = [(W1_0, b1_0, W2_0, b2_0),
            (W1_1, b1_1, W2_1, b2_1),
            (W1_2, b1_2, W2_2, b2_2)]

    xg = []
    for b in range(3):
        parts = _msgpass(x, projs[b], src3, dst3, zeros)
        x, g = _mlp_call(x, parts, *mlps[b])
        xg.append(g)

    return _final(xg[0], xg[1], xg[2], Wlin.reshape(3, D).astype(jnp.float32),
                  blin.reshape(1, 1))


# speedup vs baseline: 1.1931x; 1.1931x over previous
import functools

import numpy as np
import jax
import jax.numpy as jnp
from jax import lax
from jax.experimental import pallas as pl
from jax.experimental.pallas import tpu as pltpu
from jax.experimental.pallas import tpu_sc as plsc

N = 10000
E = 320000
D = 128
DE = 16

NC = 2
NS = 16
NW = NC * NS
EW = E // NW
CH = 80
NCH = EW // CH
NSTG = 5
CPS = NCH // NSTG
NP_ = 10240
NPS = NP_ // NS

_IDX_A = np.concatenate([np.arange(32 * g, 32 * g + 16) for g in range(4)])
_IDX_B = _IDX_A + 16

_sc_mesh = plsc.VectorSubcoreMesh(core_axis_name="c", subcore_axis_name="s")


def _msgpass_body(x_hbm, ep_hbm, src_hbm, dst_hbm, zeros_hbm, out_hbm,
                  src_v, dst_v, ep0, ep1, rows0, rows1, agg_sh,
                  sem_e0, sem_e1, sem_g0, sem_g1, sem_s0, sem_s1):
    c = lax.axis_index("c")
    s = lax.axis_index("s")
    w = c * NS + s
    pltpu.sync_copy(zeros_hbm, agg_sh.at[pl.ds(s * NPS, NPS)])
    plsc.subcore_barrier()

    def compute(ep_v, rows_v):
        shift16 = jnp.full((16,), 16, jnp.int32)
        mask_hi = jnp.full((16,), -65536, jnp.int32)

        def row(r4, carry2):
            for u in range(4):
                r = r4 * 4 + u
                for g in range(4):
                    epw = ep_v[r, pl.ds(16 * g, 16)]
                    lo = lax.bitcast_convert_type(
                        lax.shift_left(epw, shift16), jnp.float32)
                    hi = lax.bitcast_convert_type(
                        lax.bitwise_and(epw, mask_hi), jnp.float32)
                    sl_lo = pl.ds(32 * g, 16)
                    sl_hi = pl.ds(32 * g + 16, 16)
                    rows_v[r, sl_lo] = jnp.maximum(rows_v[r, sl_lo] + lo, 0.0)
                    rows_v[r, sl_hi] = jnp.maximum(rows_v[r, sl_hi] + hi, 0.0)
            return carry2

        lax.fori_loop(0, CH // 4, row, 0)

    def load(jg, jl, ep_v, rows_v, sem_e, sem_g):
        pltpu.async_copy(ep_hbm.at[w, jg], ep_v, sem_e)
        pltpu.async_copy(x_hbm.at[src_v.at[jl]], rows_v, sem_g)

    def wait_load(jg, jl, ep_v, rows_v, sem_e, sem_g):
        pltpu.make_async_copy(ep_hbm.at[w, jg], ep_v, sem_e).wait()
        pltpu.make_async_copy(x_hbm.at[src_v.at[jl]], rows_v, sem_g).wait()

    def scatter(jl, msg_v, sem_s):
        pltpu.async_copy(msg_v, agg_sh.at[dst_v.at[jl]], sem_s, add=True)

    def wait_scatter(jl, msg_v, sem_s):
        pltpu.make_async_copy(msg_v, agg_sh.at[dst_v.at[jl]], sem_s).wait()

    def stage(g, carry0):
        pltpu.sync_copy(src_hbm.at[w, g], src_v)
        pltpu.sync_copy(dst_hbm.at[w, g], dst_v)
        base = g * CPS
        load(base, 0, ep0, rows0, sem_e0, sem_g0)

        def pair(i, carry):
            j0 = 2 * i
            j1 = j0 + 1

            @pl.when(i > 0)
            def _():
                wait_scatter(j0 - 1, rows1, sem_s1)

            load(base + j1, j1, ep1, rows1, sem_e1, sem_g1)
            wait_load(base + j0, j0, ep0, rows0, sem_e0, sem_g0)
            compute(ep0, rows0)
            scatter(j0, rows0, sem_s0)

            @pl.when(i < CPS // 2)
            def _():
                wait_scatter(j0, rows0, sem_s0)
                load(base + j0 + 2, j0 + 2, ep0, rows0, sem_e0, sem_g0)

            wait_load(base + j1, j1, ep1, rows1, sem_e1, sem_g1)
            compute(ep1, rows1)
            scatter(j1, rows1, sem_s1)
            return carry

        lax.fori_loop(0, CPS // 2, pair, 0)
        jt = CPS - 1
        wait_load(base + jt, jt, ep0, rows0, sem_e0, sem_g0)
        wait_scatter(jt - 1, rows1, sem_s1)
        compute(ep0, rows0)
        scatter(jt, rows0, sem_s0)
        wait_scatter(jt, rows0, sem_s0)
        return carry0

    lax.fori_loop(0, NSTG, stage, 0)
    plsc.subcore_barrier()
    pltpu.sync_copy(agg_sh.at[pl.ds(s * NPS, NPS)],
                    out_hbm.at[c, pl.ds(s * NPS, NPS)])


_msgpass = pl.kernel(
    _msgpass_body,
    out_type=jax.ShapeDtypeStruct((NC, NP_, D), jnp.float32),
    mesh=_sc_mesh,
    scratch_types=[
        pltpu.VMEM((CPS, CH), jnp.int32),
        pltpu.VMEM((CPS, CH), jnp.int32),
        pltpu.VMEM((CH, D // 2), jnp.int32),
        pltpu.VMEM((CH, D // 2), jnp.int32),
        pltpu.VMEM((CH, D), jnp.float32),
        pltpu.VMEM((CH, D), jnp.float32),
        pltpu.VMEM_SHARED((NP_, D), jnp.float32),
        pltpu.SemaphoreType.DMA,
        pltpu.SemaphoreType.DMA,
        pltpu.SemaphoreType.DMA,
        pltpu.SemaphoreType.DMA,
        pltpu.SemaphoreType.DMA,
        pltpu.SemaphoreType.DMA,
    ],
)


EB = 6400
_DN_T = (((0,), (0,)), ((), ()))


def _pack_words(a, b):
    half = jnp.uint32(0x8000)
    au = lax.bitcast_convert_type(a, jnp.uint32) + half
    bu = lax.bitcast_convert_type(b, jnp.uint32) + half
    w = (au >> 16) | (bu & jnp.uint32(0xFFFF0000))
    return lax.bitcast_convert_type(w, jnp.int32)


def _eproj0_body(eat_ref, Wear, bear, Webr, bebr, Weu0r, beu0c,
                 p0_ref, ea1_ref):
    f32 = jnp.float32
    eat = eat_ref[...]
    a = lax.dot_general(eat, Wear[...], _DN_T,
                        preferred_element_type=f32) + bear[...]
    b = lax.dot_general(eat, Webr[...], _DN_T,
                        preferred_element_type=f32) + bebr[...]
    p0_ref[...] = _pack_words(a, b)
    ea1_ref[...] = jnp.maximum(
        lax.dot_general(Weu0r[...], eat, _DN_T, preferred_element_type=f32)
        + beu0c[...], 0.0)


def _eproj_last_body(eat_ref, Wear, bear, Webr, bebr, p_ref):
    f32 = jnp.float32
    eat = eat_ref[...]
    a = lax.dot_general(eat, Wear[...], _DN_T,
                        preferred_element_type=f32) + bear[...]
    b = lax.dot_general(eat, Webr[...], _DN_T,
                        preferred_element_type=f32) + bebr[...]
    p_ref[...] = _pack_words(a, b)


def _wspec(shape):
    return pl.BlockSpec(shape, lambda i: (0, 0))


def _eproj_call(ea, We0, be0, Weu0, beu0, We1, be1, Weu1, beu1, We2, be2):
    grid = (E // EB,)
    DH = D // 2
    wsplit = lambda We, be: (We[:, _IDX_A], be[_IDX_A].reshape(1, DH),
                             We[:, _IDX_B], be[_IDX_B].reshape(1, DH))
    tspec = pl.BlockSpec((DE, EB), lambda i: (0, i))
    wside = [_wspec((DE, DH)), _wspec((1, DH)), _wspec((DE, DH)),
             _wspec((1, DH))]
    pspec = pl.BlockSpec((EB, DH), lambda i: (i, 0))
    pshape = jax.ShapeDtypeStruct((E, DH), jnp.int32)
    tshape = jax.ShapeDtypeStruct((DE, E), jnp.float32)

    p0, ea1t = pl.pallas_call(
        _eproj0_body,
        grid=grid,
        in_specs=[tspec] + wside + [_wspec((DE, DE)), _wspec((DE, 1))],
        out_specs=[pspec, pl.BlockSpec((DE, EB), lambda i: (0, i))],
        out_shape=[pshape, tshape],
    )(ea.T, *wsplit(We0, be0), Weu0, beu0.reshape(DE, 1))
    p1, ea2t = pl.pallas_call(
        _eproj0_body,
        grid=grid,
        in_specs=[tspec] + wside + [_wspec((DE, DE)), _wspec((DE, 1))],
        out_specs=[pspec, pl.BlockSpec((DE, EB), lambda i: (0, i))],
        out_shape=[pshape, tshape],
    )(ea1t, *wsplit(We1, be1), Weu1, beu1.reshape(DE, 1))
    p2 = pl.pallas_call(
        _eproj_last_body,
        grid=grid,
        in_specs=[tspec] + wside,
        out_specs=pspec,
        out_shape=pshape,
    )(ea2t, *wsplit(We2, be2))
    return p0, p1, p2


NB = 2000


def _mlp_body(x_ref, pa_ref, pb_ref, W1r, b1r, W2r, b2r, xo_ref, xg_ref):
    f32 = jnp.float32
    i = pl.program_id(0)
    h = x_ref[...] + pa_ref[0] + pb_ref[0]
    t = jnp.maximum(jnp.dot(h, W1r[...], preferred_element_type=f32) + b1r[...],
                    0.0)
    xo = jnp.dot(t, W2r[...], preferred_element_type=f32) + b2r[...]
    xo_ref[...] = xo
    col = jnp.sum(xo, axis=0, keepdims=True)

    @pl.when(i == 0)
    def _():
        xg_ref[...] = col

    @pl.when(i > 0)
    def _():
        xg_ref[...] = xg_ref[...] + col


def _mlp_call(x, parts, W1, b1, W2, b2):
    wspec = lambda shape: pl.BlockSpec(shape, lambda i: (0, 0))
    return pl.pallas_call(
        _mlp_body,
        grid=(N // NB,),
        in_specs=[
            pl.BlockSpec((NB, D), lambda i: (i, 0)),
            pl.BlockSpec((1, NB, D), lambda i: (0, i, 0)),
            pl.BlockSpec((1, NB, D), lambda i: (1, i, 0)),
            wspec((D, D)), wspec((1, D)),
            wspec((D, D)), wspec((1, D)),
        ],
        out_specs=[
            pl.BlockSpec((NB, D), lambda i: (i, 0)),
            pl.BlockSpec((1, D), lambda i: (0, 0)),
        ],
        out_shape=[
            jax.ShapeDtypeStruct((N, D), jnp.float32),
            jax.ShapeDtypeStruct((1, D), jnp.float32),
        ],
    )(x, parts, parts, W1, b1.reshape(1, D), W2, b2.reshape(1, D))


def _final_body(g0, g1, g2, wl, bl, out_ref):
    acc = (jnp.sum(g0[...] * wl[0:1, :]) + jnp.sum(g1[...] * wl[1:2, :])
           + jnp.sum(g2[...] * wl[2:3, :]) + bl[0, 0])
    out_ref[...] = jnp.reshape(acc, (1, 1))


_final = pl.pallas_call(
    _final_body,
    out_shape=jax.ShapeDtypeStruct((1, 1), jnp.float32),
)


def kernel(x, edge_index, edge_attr,
           We0, be0, W1_0, b1_0, W2_0, b2_0, Weu0, beu0,
           We1, be1, W1_1, b1_1, W2_1, b2_1, Weu1, beu1,
           We2, be2, W1_2, b1_2, W2_2, b2_2, Weu2, beu2,
           Wlin, blin):
    src3 = edge_index[0].reshape(NW, NSTG, CPS, CH)
    dst3 = edge_index[1].reshape(NW, NSTG, CPS, CH)
    zeros = jnp.zeros((NPS, D), jnp.float32)

    p0, p1, p2 = _eproj_call(edge_attr, We0, be0, Weu0, beu0,
                             We1, be1, Weu1, beu1, We2, be2)
    projs = [p.reshape(NW, NCH, CH, D // 2) for p in (p0, p1, p2)]
    mlps = [(W1_0, b1_0, W2_0, b2_0),
            (W1_1, b1_1, W2_1, b2_1),
            (W1_2, b1_2, W2_2, b2_2)]

    xg = []
    for b in range(3):
        parts = _msgpass(x, projs[b], src3, dst3, zeros)
        x, g = _mlp_call(x, parts, *mlps[b])
        xg.append(g)

    return _final(xg[0], xg[1], xg[2], Wlin.reshape(3, D).astype(jnp.float32),
                  blin.reshape(1, 1))
